# baseline (speedup 1.0000x reference)
import jax
import jax.numpy as jnp
from jax import lax
from jax.experimental import pallas as pl
from jax.experimental.pallas import tpu as pltpu
from jax.experimental.pallas import tpu_sc as plsc

B, C, H, W = 8, 192, 112, 112
HOUT, WOUT = 224, 224
NPLANE = B * C
PIN = H * W
POUT = HOUT * WOUT
NW = 32
PLANES_PER_W = NPLANE // NW
NGRP = PIN // 16


def _unpool_body(skey_hbm, sval_hbm, out_hbm, key_v, val_v, buf_a, buf_b,
                 sem_a, sem_b):
    wid = lax.axis_index("s") * 2 + lax.axis_index("c")

    zeros16 = jnp.zeros((16,), jnp.float32)
    key_v[pl.ds(PIN, 16)] = jnp.full((16,), -1, dtype=jnp.int32)

    def zero_buf(buf):
        def zero_body(i, _):
            buf[pl.ds(i * 16, 16)] = zeros16
            return 0

        lax.fori_loop(0, POUT // 16, zero_body, 0, unroll=8)

    def load_scatter(p, buf):
        plane = wid * PLANES_PER_W + p
        seg = plane * PIN
        pltpu.sync_copy(skey_hbm.at[pl.ds(seg, PIN)], key_v.at[pl.ds(0, PIN)])
        pltpu.sync_copy(sval_hbm.at[pl.ds(seg, PIN)], val_v)
        pbase = plane * POUT

        def scat_body(j, _):
            k16 = key_v[pl.ds(j * 16, 16)]
            k17 = key_v[pl.ds(j * 16 + 1, 16)]
            vv = val_v[pl.ds(j * 16, 16)]
            local = k16 - pbase
            plsc.store_scatter(buf, [local], vv, mask=k16 != k17)
            return 0

        lax.fori_loop(0, NGRP, scat_body, 0, unroll=8)
        return plane

    def start_out(plane, buf, sem):
        pltpu.async_copy(buf, out_hbm.at[plane], sem)

    def wait_out(plane, buf, sem):
        pltpu.make_async_copy(buf, out_hbm.at[plane], sem).wait()

    zero_buf(buf_a)
    zero_buf(buf_b)
    start_out(load_scatter(0, buf_a), buf_a, sem_a)
    start_out(load_scatter(1, buf_b), buf_b, sem_b)

    def pair_body(i, _):
        pa = 2 * i
        plane_a = wid * PLANES_PER_W + pa
        wait_out(plane_a - 2, buf_a, sem_a)
        zero_buf(buf_a)
        start_out(load_scatter(pa, buf_a), buf_a, sem_a)
        plane_b = plane_a + 1
        wait_out(plane_b - 2, buf_b, sem_b)
        zero_buf(buf_b)
        start_out(load_scatter(pa + 1, buf_b), buf_b, sem_b)
        return 0

    lax.fori_loop(1, PLANES_PER_W // 2, pair_body, 0)
    last = wid * PLANES_PER_W + PLANES_PER_W
    wait_out(last - 2, buf_a, sem_a)
    wait_out(last - 1, buf_b, sem_b)


@jax.jit
def _unpool(skey, sval):
    mesh = plsc.VectorSubcoreMesh(core_axis_name="c", subcore_axis_name="s")
    return pl.kernel(
        _unpool_body,
        mesh=mesh,
        compiler_params=pltpu.CompilerParams(needs_layout_passes=False),
        out_type=jax.ShapeDtypeStruct((NPLANE, POUT), jnp.float32),
        scratch_types=[
            pltpu.VMEM((PIN + 16,), jnp.int32),
            pltpu.VMEM((PIN,), jnp.float32),
            pltpu.VMEM((POUT,), jnp.float32),
            pltpu.VMEM((POUT,), jnp.float32),
            pltpu.SemaphoreType.DMA,
            pltpu.SemaphoreType.DMA,
        ],
    )(skey, sval)


def kernel(input, indices):
    idx2d = indices.astype(jnp.int32).reshape(NPLANE, PIN)
    rows = lax.broadcasted_iota(jnp.int32, (NPLANE, PIN), 0)
    gidx = (idx2d + rows * POUT).reshape(-1)
    vals = input.reshape(-1)
    skey, sval = lax.sort_key_val(gidx, vals, is_stable=False)
    out = _unpool(skey, sval)
    return out.reshape(B, C, HOUT, WOUT)


# speedup vs baseline: 1.0013x; 1.0013x over previous
import jax
import jax.numpy as jnp
from jax import lax
from jax.experimental import pallas as pl
from jax.experimental.pallas import tpu as pltpu
from jax.experimental.pallas import tpu_sc as plsc

B, C, H, W = 8, 192, 112, 112
HOUT, WOUT = 224, 224
NPLANE = B * C
PIN = H * W
POUT = HOUT * WOUT
NW = 32
PLANES_PER_W = NPLANE // NW
NGRP = PIN // 16


def _unpool_body(skey_hbm, sval_hbm, out_hbm, key_v, val_v, buf_a, buf_b,
                 sem_a, sem_b, sem_in):
    wid = lax.axis_index("s") * 2 + lax.axis_index("c")

    zeros16 = jnp.zeros((16,), jnp.float32)
    key_v[pl.ds(PIN, 16)] = jnp.full((16,), -1, dtype=jnp.int32)

    def zero_buf(buf):
        def zero_body(i, _):
            buf[pl.ds(i * 16, 16)] = zeros16
            return 0

        lax.fori_loop(0, POUT // 16, zero_body, 0, unroll=16)

    def load_scatter(p, buf):
        plane = wid * PLANES_PER_W + p
        seg = plane * PIN
        pltpu.async_copy(skey_hbm.at[pl.ds(seg, PIN)], key_v.at[pl.ds(0, PIN)],
                         sem_in)
        pltpu.async_copy(sval_hbm.at[pl.ds(seg, PIN)], val_v, sem_in)
        pltpu.make_async_copy(skey_hbm.at[pl.ds(seg, PIN)],
                              key_v.at[pl.ds(0, PIN)], sem_in).wait()
        pltpu.make_async_copy(sval_hbm.at[pl.ds(seg, PIN)], val_v,
                              sem_in).wait()
        pbase = plane * POUT

        def scat_body(j, _):
            k16 = key_v[pl.ds(j * 16, 16)]
            k17 = key_v[pl.ds(j * 16 + 1, 16)]
            vv = val_v[pl.ds(j * 16, 16)]
            local = k16 - pbase
            plsc.store_scatter(buf, [local], vv, mask=k16 != k17)
            return 0

        lax.fori_loop(0, NGRP, scat_body, 0, unroll=16)
        return plane

    def start_out(plane, buf, sem):
        pltpu.async_copy(buf, out_hbm.at[plane], sem)

    def wait_out(plane, buf, sem):
        pltpu.make_async_copy(buf, out_hbm.at[plane], sem).wait()

    zero_buf(buf_a)
    zero_buf(buf_b)
    start_out(load_scatter(0, buf_a), buf_a, sem_a)
    start_out(load_scatter(1, buf_b), buf_b, sem_b)

    def pair_body(i, _):
        pa = 2 * i
        plane_a = wid * PLANES_PER_W + pa
        wait_out(plane_a - 2, buf_a, sem_a)
        zero_buf(buf_a)
        start_out(load_scatter(pa, buf_a), buf_a, sem_a)
        plane_b = plane_a + 1
        wait_out(plane_b - 2, buf_b, sem_b)
        zero_buf(buf_b)
        start_out(load_scatter(pa + 1, buf_b), buf_b, sem_b)
        return 0

    lax.fori_loop(1, PLANES_PER_W // 2, pair_body, 0)
    last = wid * PLANES_PER_W + PLANES_PER_W
    wait_out(last - 2, buf_a, sem_a)
    wait_out(last - 1, buf_b, sem_b)


@jax.jit
def _unpool(skey, sval):
    mesh = plsc.VectorSubcoreMesh(core_axis_name="c", subcore_axis_name="s")
    return pl.kernel(
        _unpool_body,
        mesh=mesh,
        compiler_params=pltpu.CompilerParams(needs_layout_passes=False),
        out_type=jax.ShapeDtypeStruct((NPLANE, POUT), jnp.float32),
        scratch_types=[
            pltpu.VMEM((PIN + 16,), jnp.int32),
            pltpu.VMEM((PIN,), jnp.float32),
            pltpu.VMEM((POUT,), jnp.float32),
            pltpu.VMEM((POUT,), jnp.float32),
            pltpu.SemaphoreType.DMA,
            pltpu.SemaphoreType.DMA,
            pltpu.SemaphoreType.DMA,
        ],
    )(skey, sval)


def kernel(input, indices):
    idx2d = indices.astype(jnp.int32).reshape(NPLANE, PIN)
    rows = lax.broadcasted_iota(jnp.int32, (NPLANE, PIN), 0)
    gidx = (idx2d + rows * POUT).reshape(-1)
    vals = input.reshape(-1)
    skey, sval = lax.sort_key_val(gidx, vals, is_stable=False)
    out = _unpool(skey, sval)
    return out.reshape(B, C, HOUT, WOUT)


# speedup vs baseline: 1.0042x; 1.0029x over previous
import jax
import jax.numpy as jnp
from jax import lax
from jax.experimental import pallas as pl
from jax.experimental.pallas import tpu as pltpu
from jax.experimental.pallas import tpu_sc as plsc

B, C, H, W = 8, 192, 112, 112
HOUT, WOUT = 224, 224
NPLANE = B * C
PIN = H * W
POUT = HOUT * WOUT
NW = 32
PLANES_PER_W = NPLANE // NW
NGRP = PIN // 16


def _unpool_body(skey_hbm, sval_hbm, out_hbm, key_v, val_v, buf_a, buf_b,
                 sem_a, sem_b, sem_in):
    wid = lax.axis_index("s") * 2 + lax.axis_index("c")

    zeros16 = jnp.zeros((16,), jnp.float32)
    key_v[pl.ds(PIN, 16)] = jnp.full((16,), -1, dtype=jnp.int32)

    def zero_buf(buf):
        def zero_body(i, _):
            buf[pl.ds(i * 16, 16)] = zeros16
            return 0

        lax.fori_loop(0, POUT // 16, zero_body, 0, unroll=16)

    def start_in(p):
        seg = (wid * PLANES_PER_W + p) * PIN
        pltpu.async_copy(skey_hbm.at[pl.ds(seg, PIN)], key_v.at[pl.ds(0, PIN)],
                         sem_in)
        pltpu.async_copy(sval_hbm.at[pl.ds(seg, PIN)], val_v, sem_in)

    def do_scatter(p, buf):
        plane = wid * PLANES_PER_W + p
        seg = plane * PIN
        pltpu.make_async_copy(skey_hbm.at[pl.ds(seg, PIN)],
                              key_v.at[pl.ds(0, PIN)], sem_in).wait()
        pltpu.make_async_copy(sval_hbm.at[pl.ds(seg, PIN)], val_v,
                              sem_in).wait()
        pbase = plane * POUT

        def scat_body(j, _):
            k16 = key_v[pl.ds(j * 16, 16)]
            k17 = key_v[pl.ds(j * 16 + 1, 16)]
            vv = val_v[pl.ds(j * 16, 16)]
            local = k16 - pbase
            plsc.store_scatter(buf, [local], vv, mask=k16 != k17)
            return 0

        lax.fori_loop(0, NGRP, scat_body, 0, unroll=16)
        return plane

    def start_out(plane, buf, sem):
        pltpu.async_copy(buf, out_hbm.at[plane], sem)

    def wait_out(plane, buf, sem):
        pltpu.make_async_copy(buf, out_hbm.at[plane], sem).wait()

    start_in(0)
    zero_buf(buf_a)
    zero_buf(buf_b)
    start_out(do_scatter(0, buf_a), buf_a, sem_a)
    start_in(1)
    start_out(do_scatter(1, buf_b), buf_b, sem_b)

    def pair_body(i, _):
        pa = 2 * i
        plane_a = wid * PLANES_PER_W + pa
        start_in(pa)
        wait_out(plane_a - 2, buf_a, sem_a)
        zero_buf(buf_a)
        start_out(do_scatter(pa, buf_a), buf_a, sem_a)
        plane_b = plane_a + 1
        start_in(pa + 1)
        wait_out(plane_b - 2, buf_b, sem_b)
        zero_buf(buf_b)
        start_out(do_scatter(pa + 1, buf_b), buf_b, sem_b)
        return 0

    lax.fori_loop(1, PLANES_PER_W // 2, pair_body, 0)
    last = wid * PLANES_PER_W + PLANES_PER_W
    wait_out(last - 2, buf_a, sem_a)
    wait_out(last - 1, buf_b, sem_b)


@jax.jit
def _unpool(skey, sval):
    mesh = plsc.VectorSubcoreMesh(core_axis_name="c", subcore_axis_name="s")
    return pl.kernel(
        _unpool_body,
        mesh=mesh,
        compiler_params=pltpu.CompilerParams(needs_layout_passes=False),
        out_type=jax.ShapeDtypeStruct((NPLANE, POUT), jnp.float32),
        scratch_types=[
            pltpu.VMEM((PIN + 16,), jnp.int32),
            pltpu.VMEM((PIN,), jnp.float32),
            pltpu.VMEM((POUT,), jnp.float32),
            pltpu.VMEM((POUT,), jnp.float32),
            pltpu.SemaphoreType.DMA,
            pltpu.SemaphoreType.DMA,
            pltpu.SemaphoreType.DMA,
        ],
    )(skey, sval)


def kernel(input, indices):
    idx2d = indices.astype(jnp.int32).reshape(NPLANE, PIN)
    rows = lax.broadcasted_iota(jnp.int32, (NPLANE, PIN), 0)
    gidx = (idx2d + rows * POUT).reshape(-1)
    vals = input.reshape(-1)
    skey, sval = lax.sort_key_val(gidx, vals, is_stable=False)
    out = _unpool(skey, sval)
    return out.reshape(B, C, HOUT, WOUT)
